# baseline (speedup 1.0000x reference)
import jax
import jax.numpy as jnp
from jax import lax
from jax.experimental import pallas as pl
from jax.experimental.pallas import tpu as pltpu
from jax.experimental.pallas import tpu_sc as plsc

N_NODES = 10000
N_EDGES = 320000
NC = 2
NS = 16
NW = NC * NS
CHUNK = 96
GRP = 8
N_CHUNKS = 112
N_GRPS = N_CHUNKS // GRP
N_PAIRS = N_GRPS // 2
EPT_PAD = N_CHUNKS * CHUNK
N_PAD = 10240
TRASH = N_PAD - 1
ROWS_PER_TILE = N_PAD // NS
WCHUNK = 80
F = 128

_MESH = plsc.VectorSubcoreMesh(core_axis_name="c", subcore_axis_name="s")


def _fill_rows(ref, nrows, value):
    def body(i, _):
        for j in range(F // 16):
            ref[i, pl.ds(j * 16, 16)] = jnp.full((16,), value, jnp.float32)
        return 0
    lax.fori_loop(0, nrows, body, 0)


def _zero_accum(s, stage, acc_sh):
    _fill_rows(stage, WCHUNK, 0.0)
    for k in range(ROWS_PER_TILE // WCHUNK):
        r0 = s * ROWS_PER_TILE + k * WCHUNK
        pltpu.sync_copy(stage.at[pl.ds(0, WCHUNK)], acc_sh.at[pl.ds(r0, WCHUNK)])


def _writeout(c, s, stage, acc_sh, out_hbm):
    for k in range(ROWS_PER_TILE // WCHUNK):
        r0 = s * ROWS_PER_TILE + k * WCHUNK
        pltpu.sync_copy(acc_sh.at[pl.ds(r0, WCHUNK)], stage.at[pl.ds(0, WCHUNK)])
        pltpu.sync_copy(stage.at[pl.ds(0, WCHUNK)],
                        out_hbm.at[c, pl.ds(r0, WCHUNK)])


def _pad_edges(src, dst):
    pad = EPT_PAD - N_EDGES // NW
    srcp = jnp.concatenate(
        [src.reshape(NW, N_EDGES // NW),
         jnp.zeros((NW, pad), src.dtype)], axis=1).reshape(NW * N_CHUNKS,
                                                           CHUNK)
    dstp = jnp.concatenate(
        [dst.reshape(NW, N_EDGES // NW),
         jnp.full((NW, pad), TRASH, dst.dtype)], axis=1)
    return srcp, dstp.reshape(NW * N_CHUNKS, CHUNK)


def _sc_agg(h, srcp, dstp):
    N_PAIRS_C = N_CHUNKS // 2

    def body(h_hbm, src_hbm, dst_hbm, agg_out, agg_sh, rows_a, rows_b,
             src_v0, src_v1, dst_v0, dst_v1, sga, sgb):
        c = lax.axis_index("c")
        s = lax.axis_index("s")
        wid = s * NC + c

        _zero_accum(s, rows_a, agg_sh)
        plsc.subcore_barrier()

        rbase = wid * N_CHUNKS

        pltpu.sync_copy(src_hbm.at[rbase], src_v0)
        pltpu.sync_copy(dst_hbm.at[rbase], dst_v0)
        pltpu.async_copy(h_hbm.at[src_v0], rows_a, sga)

        def pair_body(j, _):
            c0 = rbase + 2 * j
            pltpu.sync_copy(src_hbm.at[c0 + 1], src_v1)
            pltpu.sync_copy(dst_hbm.at[c0 + 1], dst_v1)
            pltpu.async_copy(h_hbm.at[src_v1], rows_b, sgb)
            pltpu.make_async_copy(h_hbm.at[src_v0], rows_a, sga).wait()
            pltpu.sync_copy(rows_a, agg_sh.at[dst_v0], add=True)

            def ahead():
                pltpu.sync_copy(src_hbm.at[c0 + 2], src_v0)
                pltpu.sync_copy(dst_hbm.at[c0 + 2], dst_v0)
                pltpu.async_copy(h_hbm.at[src_v0], rows_a, sga)
            pl.when(j < N_PAIRS_C - 1)(ahead)
            pltpu.make_async_copy(h_hbm.at[src_v1], rows_b, sgb).wait()
            pltpu.sync_copy(rows_b, agg_sh.at[dst_v1], add=True)
            return 0

        lax.fori_loop(0, N_PAIRS_C, pair_body, 0)
        plsc.subcore_barrier()
        _writeout(c, s, rows_a, agg_sh, agg_out)

    return pl.kernel(
        body,
        out_type=jax.ShapeDtypeStruct((NC, N_PAD, F), jnp.float32),
        mesh=_MESH,
        scratch_types=(
            pltpu.VMEM_SHARED((N_PAD, F), jnp.float32),
            pltpu.VMEM((CHUNK, F), jnp.float32),
            pltpu.VMEM((CHUNK, F), jnp.float32),
            pltpu.VMEM((CHUNK,), jnp.int32),
            pltpu.VMEM((CHUNK,), jnp.int32),
            pltpu.VMEM((CHUNK,), jnp.int32),
            pltpu.VMEM((CHUNK,), jnp.int32),
            pltpu.SemaphoreType.DMA,
            pltpu.SemaphoreType.DMA,
        ),
    )(h, srcp, dstp)


def _sc_deg(dstp):
    def body(dst_hbm, deg_out, deg_sh, ones_v, dstg_a, dstg_b,
             ssa, ssb, sad, sbd):
        c = lax.axis_index("c")
        s = lax.axis_index("s")
        wid = s * NC + c

        _zero_accum(s, ones_v, deg_sh)
        _fill_rows(ones_v, CHUNK, 1.0)
        plsc.subcore_barrier()

        rbase = wid * N_CHUNKS

        def issue_idx(g, dstg, sd):
            pltpu.async_copy(dst_hbm.at[pl.ds(rbase + g * GRP, GRP)],
                             dstg, sd)

        def wait_idx(dstg, sd):
            pltpu.make_async_copy(dst_hbm.at[pl.ds(0, GRP)], dstg, sd).wait()

        def scatter_group(dstg, sem):
            for k in range(GRP):
                pltpu.async_copy(ones_v, deg_sh.at[dstg.at[k]], sem,
                                 add=True)

        def drain_group(dstg, sem):
            for k in range(GRP):
                pltpu.make_async_copy(ones_v, deg_sh.at[dstg.at[k]],
                                      sem).wait()

        pltpu.sync_copy(dst_hbm.at[pl.ds(rbase, GRP)], dstg_a)

        def pair_body(p, _):
            ga = 2 * p
            issue_idx(ga + 1, dstg_b, sbd)
            scatter_group(dstg_a, ssa)
            wait_idx(dstg_b, sbd)
            drain_group(dstg_a, ssa)
            not_last = p < N_PAIRS - 1
            pl.when(not_last)(lambda: issue_idx(ga + 2, dstg_a, sad))
            scatter_group(dstg_b, ssb)
            pl.when(not_last)(lambda: wait_idx(dstg_a, sad))
            drain_group(dstg_b, ssb)
            return 0

        lax.fori_loop(0, N_PAIRS, pair_body, 0)
        plsc.subcore_barrier()
        _writeout(c, s, ones_v, deg_sh, deg_out)

    return pl.kernel(
        body,
        out_type=jax.ShapeDtypeStruct((NC, N_PAD, F), jnp.float32),
        mesh=_MESH,
        scratch_types=(
            pltpu.VMEM_SHARED((N_PAD, F), jnp.float32),
            pltpu.VMEM((CHUNK, F), jnp.float32),
            pltpu.VMEM((GRP, CHUNK), jnp.int32),
            pltpu.VMEM((GRP, CHUNK), jnp.int32),
            pltpu.SemaphoreType.DMA,
            pltpu.SemaphoreType.DMA,
            pltpu.SemaphoreType.DMA,
            pltpu.SemaphoreType.DMA,
        ),
    )(dstp)


def _tc_layer(h, a0, a1, d0, d1, Ws, Wn, b, act):
    n, din = h.shape
    dout = Ws.shape[1]
    blk = 1000

    def body(h_ref, a0_ref, a1_ref, d0_ref, d1_ref, ws_ref, wn_ref, b_ref,
             o_ref):
        deg = d0_ref[:, 0:1] + d1_ref[:, 0:1]
        inv = 1.0 / jnp.maximum(deg, 1.0)
        hn = (a0_ref[...] + a1_ref[...]) * inv
        out = jnp.dot(h_ref[...], ws_ref[...],
                      preferred_element_type=jnp.float32)
        out = out + jnp.dot(hn, wn_ref[...],
                            preferred_element_type=jnp.float32)
        out = out + b_ref[...]
        if act:
            out = jnp.maximum(out, 0.0)
        o_ref[...] = out

    return pl.pallas_call(
        body,
        grid=(n // blk,),
        in_specs=[
            pl.BlockSpec((blk, din), lambda i: (i, 0)),
            pl.BlockSpec((blk, din), lambda i: (i, 0)),
            pl.BlockSpec((blk, din), lambda i: (i, 0)),
            pl.BlockSpec((blk, F), lambda i: (i, 0)),
            pl.BlockSpec((blk, F), lambda i: (i, 0)),
            pl.BlockSpec((din, dout), lambda i: (0, 0)),
            pl.BlockSpec((din, dout), lambda i: (0, 0)),
            pl.BlockSpec((1, dout), lambda i: (0, 0)),
        ],
        out_specs=pl.BlockSpec((blk, dout), lambda i: (i, 0)),
        out_shape=jax.ShapeDtypeStruct((n, dout), jnp.float32),
    )(h, a0, a1, d0, d1, Ws, Wn, b.reshape(1, dout))


def kernel(features, edge_index, Ws0, Wn0, b0, Ws1, Wn1, b1, Ws2, Wn2, b2):
    src = edge_index[0]
    dst = edge_index[1]
    srcp, dstp = _pad_edges(src, dst)

    degp = _sc_deg(dstp)
    d0, d1 = degp[0], degp[1]
    agg0 = _sc_agg(features, srcp, dstp)
    h1 = _tc_layer(features, agg0[0], agg0[1], d0, d1, Ws0, Wn0, b0, True)
    agg1 = _sc_agg(h1, srcp, dstp)
    h2 = _tc_layer(h1, agg1[0], agg1[1], d0, d1, Ws1, Wn1, b1, True)
    agg2 = _sc_agg(h2, srcp, dstp)
    return _tc_layer(h2, agg2[0], agg2[1], d0, d1, Ws2, Wn2, b2, False)


# speedup vs baseline: 3.0857x; 3.0857x over previous
import jax
import jax.numpy as jnp
from jax import lax
from jax.experimental import pallas as pl
from jax.experimental.pallas import tpu as pltpu
from jax.experimental.pallas import tpu_sc as plsc

N_NODES = 10000
N_EDGES = 320000
NC = 2
NS = 16
NW = NC * NS
CHUNK = 96
GRP = 8
N_CHUNKS = 112
N_GRPS = N_CHUNKS // GRP
N_PAIRS = N_GRPS // 2
EPT_PAD = N_CHUNKS * CHUNK
N_PAD = 10240
TRASH = N_PAD - 1
ROWS_PER_TILE = N_PAD // NS
WCHUNK = 80
F = 128

_MESH = plsc.VectorSubcoreMesh(core_axis_name="c", subcore_axis_name="s")


def _fill_rows(ref, nrows, value):
    def body(i, _):
        for j in range(F // 16):
            ref[i, pl.ds(j * 16, 16)] = jnp.full((16,), value, jnp.float32)
        return 0
    lax.fori_loop(0, nrows, body, 0)


def _zero_accum(s, stage, acc_sh):
    _fill_rows(stage, WCHUNK, 0.0)
    for k in range(ROWS_PER_TILE // WCHUNK):
        r0 = s * ROWS_PER_TILE + k * WCHUNK
        pltpu.sync_copy(stage.at[pl.ds(0, WCHUNK)], acc_sh.at[pl.ds(r0, WCHUNK)])


def _writeout(c, s, stage, acc_sh, out_hbm):
    for k in range(ROWS_PER_TILE // WCHUNK):
        r0 = s * ROWS_PER_TILE + k * WCHUNK
        pltpu.sync_copy(acc_sh.at[pl.ds(r0, WCHUNK)], stage.at[pl.ds(0, WCHUNK)])
        pltpu.sync_copy(stage.at[pl.ds(0, WCHUNK)],
                        out_hbm.at[c, pl.ds(r0, WCHUNK)])


def _pad_edges(src, dst):
    pad = EPT_PAD - N_EDGES // NW
    srcp = jnp.concatenate(
        [src.reshape(NW, N_EDGES // NW),
         jnp.zeros((NW, pad), src.dtype)], axis=1).reshape(NW * N_CHUNKS,
                                                           CHUNK)
    dstp = jnp.concatenate(
        [dst.reshape(NW, N_EDGES // NW),
         jnp.full((NW, pad), TRASH, dst.dtype)], axis=1)
    return srcp, dstp.reshape(NW * N_CHUNKS, CHUNK)


AGG_CHUNK = 80
AGG_N_CHUNKS = (N_EDGES // NW) // AGG_CHUNK


def _sc_agg(h, src, dst):
    def body(h_hbm, src_hbm, dst_hbm, agg_out, agg_sh, src_v, dst_v, rows_v,
             sem):
        c = lax.axis_index("c")
        s = lax.axis_index("s")
        wid = s * NC + c

        _zero_accum(s, rows_v, agg_sh)
        plsc.subcore_barrier()

        def ebody(i, _):
            base = wid * (N_EDGES // NW) + i * AGG_CHUNK
            pltpu.sync_copy(src_hbm.at[pl.ds(base, AGG_CHUNK)], src_v)
            pltpu.sync_copy(dst_hbm.at[pl.ds(base, AGG_CHUNK)], dst_v)
            pltpu.async_copy(h_hbm.at[src_v], rows_v, sem).wait()
            pltpu.sync_copy(rows_v, agg_sh.at[dst_v], add=True)
            return 0
        lax.fori_loop(0, AGG_N_CHUNKS, ebody, 0)
        plsc.subcore_barrier()
        _writeout(c, s, rows_v, agg_sh, agg_out)

    return pl.kernel(
        body,
        out_type=jax.ShapeDtypeStruct((NC, N_PAD, F), jnp.float32),
        mesh=_MESH,
        scratch_types=(
            pltpu.VMEM_SHARED((N_PAD, F), jnp.float32),
            pltpu.VMEM((AGG_CHUNK,), jnp.int32),
            pltpu.VMEM((AGG_CHUNK,), jnp.int32),
            pltpu.VMEM((AGG_CHUNK, F), jnp.float32),
            pltpu.SemaphoreType.DMA,
        ),
    )(h, src, dst)


def _sc_deg(dstp):
    def body(dst_hbm, deg_out, deg_sh, ones_v, dstg_a, dstg_b,
             ssa, ssb, sad, sbd):
        c = lax.axis_index("c")
        s = lax.axis_index("s")
        wid = s * NC + c

        _zero_accum(s, ones_v, deg_sh)
        _fill_rows(ones_v, CHUNK, 1.0)
        plsc.subcore_barrier()

        rbase = wid * N_CHUNKS

        def issue_idx(g, dstg, sd):
            pltpu.async_copy(dst_hbm.at[pl.ds(rbase + g * GRP, GRP)],
                             dstg, sd)

        def wait_idx(dstg, sd):
            pltpu.make_async_copy(dst_hbm.at[pl.ds(0, GRP)], dstg, sd).wait()

        def scatter_group(dstg, sem):
            for k in range(GRP):
                pltpu.async_copy(ones_v, deg_sh.at[dstg.at[k]], sem,
                                 add=True)

        def drain_group(dstg, sem):
            for k in range(GRP):
                pltpu.make_async_copy(ones_v, deg_sh.at[dstg.at[k]],
                                      sem).wait()

        pltpu.sync_copy(dst_hbm.at[pl.ds(rbase, GRP)], dstg_a)

        def pair_body(p, _):
            ga = 2 * p
            issue_idx(ga + 1, dstg_b, sbd)
            scatter_group(dstg_a, ssa)
            wait_idx(dstg_b, sbd)
            drain_group(dstg_a, ssa)
            not_last = p < N_PAIRS - 1
            pl.when(not_last)(lambda: issue_idx(ga + 2, dstg_a, sad))
            scatter_group(dstg_b, ssb)
            pl.when(not_last)(lambda: wait_idx(dstg_a, sad))
            drain_group(dstg_b, ssb)
            return 0

        lax.fori_loop(0, N_PAIRS, pair_body, 0)
        plsc.subcore_barrier()
        _writeout(c, s, ones_v, deg_sh, deg_out)

    return pl.kernel(
        body,
        out_type=jax.ShapeDtypeStruct((NC, N_PAD, F), jnp.float32),
        mesh=_MESH,
        scratch_types=(
            pltpu.VMEM_SHARED((N_PAD, F), jnp.float32),
            pltpu.VMEM((CHUNK, F), jnp.float32),
            pltpu.VMEM((GRP, CHUNK), jnp.int32),
            pltpu.VMEM((GRP, CHUNK), jnp.int32),
            pltpu.SemaphoreType.DMA,
            pltpu.SemaphoreType.DMA,
            pltpu.SemaphoreType.DMA,
            pltpu.SemaphoreType.DMA,
        ),
    )(dstp)


def _tc_layer(h, a0, a1, d0, d1, Ws, Wn, b, act):
    n, din = h.shape
    dout = Ws.shape[1]
    blk = 1000

    def body(h_ref, a0_ref, a1_ref, d0_ref, d1_ref, ws_ref, wn_ref, b_ref,
             o_ref):
        deg = d0_ref[:, 0:1] + d1_ref[:, 0:1]
        inv = 1.0 / jnp.maximum(deg, 1.0)
        hn = (a0_ref[...] + a1_ref[...]) * inv
        out = jnp.dot(h_ref[...], ws_ref[...],
                      preferred_element_type=jnp.float32)
        out = out + jnp.dot(hn, wn_ref[...],
                            preferred_element_type=jnp.float32)
        out = out + b_ref[...]
        if act:
            out = jnp.maximum(out, 0.0)
        o_ref[...] = out

    return pl.pallas_call(
        body,
        grid=(n // blk,),
        in_specs=[
            pl.BlockSpec((blk, din), lambda i: (i, 0)),
            pl.BlockSpec((blk, din), lambda i: (i, 0)),
            pl.BlockSpec((blk, din), lambda i: (i, 0)),
            pl.BlockSpec((blk, F), lambda i: (i, 0)),
            pl.BlockSpec((blk, F), lambda i: (i, 0)),
            pl.BlockSpec((din, dout), lambda i: (0, 0)),
            pl.BlockSpec((din, dout), lambda i: (0, 0)),
            pl.BlockSpec((1, dout), lambda i: (0, 0)),
        ],
        out_specs=pl.BlockSpec((blk, dout), lambda i: (i, 0)),
        out_shape=jax.ShapeDtypeStruct((n, dout), jnp.float32),
    )(h, a0, a1, d0, d1, Ws, Wn, b.reshape(1, dout))


def kernel(features, edge_index, Ws0, Wn0, b0, Ws1, Wn1, b1, Ws2, Wn2, b2):
    src = edge_index[0]
    dst = edge_index[1]
    srcp, dstp = _pad_edges(src, dst)

    degp = _sc_deg(dstp)
    d0, d1 = degp[0], degp[1]
    agg0 = _sc_agg(features, src, dst)
    h1 = _tc_layer(features, agg0[0], agg0[1], d0, d1, Ws0, Wn0, b0, True)
    agg1 = _sc_agg(h1, src, dst)
    h2 = _tc_layer(h1, agg1[0], agg1[1], d0, d1, Ws1, Wn1, b1, True)
    agg2 = _sc_agg(h2, src, dst)
    return _tc_layer(h2, agg2[0], agg2[1], d0, d1, Ws2, Wn2, b2, False)


# speedup vs baseline: 4.2653x; 1.3823x over previous
import jax
import jax.numpy as jnp
from jax import lax
from jax.experimental import pallas as pl
from jax.experimental.pallas import tpu as pltpu
from jax.experimental.pallas import tpu_sc as plsc

N_NODES = 10000
N_EDGES = 320000
NC = 2
NS = 16
NW = NC * NS
CHUNK = 96
GRP = 8
N_CHUNKS = 112
N_GRPS = N_CHUNKS // GRP
N_PAIRS = N_GRPS // 2
EPT_PAD = N_CHUNKS * CHUNK
N_PAD = 10240
TRASH = N_PAD - 1
ROWS_PER_TILE = N_PAD // NS
WCHUNK = 80
F = 128

_MESH = plsc.VectorSubcoreMesh(core_axis_name="c", subcore_axis_name="s")


def _fill_rows(ref, nrows, value):
    def body(i, _):
        for j in range(F // 16):
            ref[i, pl.ds(j * 16, 16)] = jnp.full((16,), value, jnp.float32)
        return 0
    lax.fori_loop(0, nrows, body, 0)


def _zero_accum(s, stage, acc_sh):
    _fill_rows(stage, WCHUNK, 0.0)
    for k in range(ROWS_PER_TILE // WCHUNK):
        r0 = s * ROWS_PER_TILE + k * WCHUNK
        pltpu.sync_copy(stage.at[pl.ds(0, WCHUNK)], acc_sh.at[pl.ds(r0, WCHUNK)])


def _writeout(c, s, stage, acc_sh, out_hbm):
    for k in range(ROWS_PER_TILE // WCHUNK):
        r0 = s * ROWS_PER_TILE + k * WCHUNK
        pltpu.sync_copy(acc_sh.at[pl.ds(r0, WCHUNK)], stage.at[pl.ds(0, WCHUNK)])
        pltpu.sync_copy(stage.at[pl.ds(0, WCHUNK)],
                        out_hbm.at[c, pl.ds(r0, WCHUNK)])


def _pad_edges(src, dst):
    pad = EPT_PAD - N_EDGES // NW
    srcp = jnp.concatenate(
        [src.reshape(NW, N_EDGES // NW),
         jnp.zeros((NW, pad), src.dtype)], axis=1).reshape(NW * N_CHUNKS,
                                                           CHUNK)
    dstp = jnp.concatenate(
        [dst.reshape(NW, N_EDGES // NW),
         jnp.full((NW, pad), TRASH, dst.dtype)], axis=1)
    return srcp, dstp.reshape(NW * N_CHUNKS, CHUNK)


AGG_CHUNK = 80
AGG_N_CHUNKS = (N_EDGES // NW) // AGG_CHUNK


def _sc_agg(h, src, dst):
    EPT = N_EDGES // NW
    NPAIR = (AGG_N_CHUNKS - 1) // 2

    def body(h_hbm, src_hbm, dst_hbm, agg_out, agg_sh,
             src_v0, dst_v0, src_v1, dst_v1, rows_v,
             sg, s0s, s0d, s1s, s1d):
        c = lax.axis_index("c")
        s = lax.axis_index("s")
        wid = s * NC + c

        _zero_accum(s, rows_v, agg_sh)
        plsc.subcore_barrier()

        ebase = wid * EPT

        def load_idx(i, sv, dv, ss, sd):
            b = ebase + i * AGG_CHUNK
            pltpu.async_copy(src_hbm.at[pl.ds(b, AGG_CHUNK)], sv, ss)
            pltpu.async_copy(dst_hbm.at[pl.ds(b, AGG_CHUNK)], dv, sd)

        def wait_idx(sv, dv, ss, sd):
            pltpu.make_async_copy(src_hbm.at[pl.ds(0, AGG_CHUNK)], sv,
                                  ss).wait()
            pltpu.make_async_copy(dst_hbm.at[pl.ds(0, AGG_CHUNK)], dv,
                                  sd).wait()

        def do_chunk(sv, dv):
            pltpu.async_copy(h_hbm.at[sv], rows_v, sg).wait()
            pltpu.sync_copy(rows_v, agg_sh.at[dv], add=True)

        load_idx(0, src_v0, dst_v0, s0s, s0d)

        def pair_body(j, _):
            c0 = 2 * j
            load_idx(c0 + 1, src_v1, dst_v1, s1s, s1d)
            wait_idx(src_v0, dst_v0, s0s, s0d)
            do_chunk(src_v0, dst_v0)
            load_idx(c0 + 2, src_v0, dst_v0, s0s, s0d)
            wait_idx(src_v1, dst_v1, s1s, s1d)
            do_chunk(src_v1, dst_v1)
            return 0

        lax.fori_loop(0, NPAIR, pair_body, 0)
        wait_idx(src_v0, dst_v0, s0s, s0d)
        do_chunk(src_v0, dst_v0)
        plsc.subcore_barrier()
        _writeout(c, s, rows_v, agg_sh, agg_out)

    return pl.kernel(
        body,
        out_type=jax.ShapeDtypeStruct((NC, N_PAD, F), jnp.float32),
        mesh=_MESH,
        scratch_types=(
            pltpu.VMEM_SHARED((N_PAD, F), jnp.float32),
            pltpu.VMEM((AGG_CHUNK,), jnp.int32),
            pltpu.VMEM((AGG_CHUNK,), jnp.int32),
            pltpu.VMEM((AGG_CHUNK,), jnp.int32),
            pltpu.VMEM((AGG_CHUNK,), jnp.int32),
            pltpu.VMEM((AGG_CHUNK, F), jnp.float32),
            pltpu.SemaphoreType.DMA,
            pltpu.SemaphoreType.DMA,
            pltpu.SemaphoreType.DMA,
            pltpu.SemaphoreType.DMA,
            pltpu.SemaphoreType.DMA,
        ),
    )(h, src, dst)


def _sc_deg(dstp):
    def body(dst_hbm, deg_out, deg_sh, ones_v, dstg_a, dstg_b,
             ssa, ssb, sad, sbd):
        c = lax.axis_index("c")
        s = lax.axis_index("s")
        wid = s * NC + c

        _zero_accum(s, ones_v, deg_sh)
        _fill_rows(ones_v, CHUNK, 1.0)
        plsc.subcore_barrier()

        rbase = wid * N_CHUNKS

        def issue_idx(g, dstg, sd):
            pltpu.async_copy(dst_hbm.at[pl.ds(rbase + g * GRP, GRP)],
                             dstg, sd)

        def wait_idx(dstg, sd):
            pltpu.make_async_copy(dst_hbm.at[pl.ds(0, GRP)], dstg, sd).wait()

        def scatter_group(dstg, sem):
            for k in range(GRP):
                pltpu.async_copy(ones_v, deg_sh.at[dstg.at[k]], sem,
                                 add=True)

        def drain_group(dstg, sem):
            for k in range(GRP):
                pltpu.make_async_copy(ones_v, deg_sh.at[dstg.at[k]],
                                      sem).wait()

        pltpu.sync_copy(dst_hbm.at[pl.ds(rbase, GRP)], dstg_a)

        def pair_body(p, _):
            ga = 2 * p
            issue_idx(ga + 1, dstg_b, sbd)
            scatter_group(dstg_a, ssa)
            wait_idx(dstg_b, sbd)
            drain_group(dstg_a, ssa)
            not_last = p < N_PAIRS - 1
            pl.when(not_last)(lambda: issue_idx(ga + 2, dstg_a, sad))
            scatter_group(dstg_b, ssb)
            pl.when(not_last)(lambda: wait_idx(dstg_a, sad))
            drain_group(dstg_b, ssb)
            return 0

        lax.fori_loop(0, N_PAIRS, pair_body, 0)
        plsc.subcore_barrier()
        _writeout(c, s, ones_v, deg_sh, deg_out)

    return pl.kernel(
        body,
        out_type=jax.ShapeDtypeStruct((NC, N_PAD, F), jnp.float32),
        mesh=_MESH,
        scratch_types=(
            pltpu.VMEM_SHARED((N_PAD, F), jnp.float32),
            pltpu.VMEM((CHUNK, F), jnp.float32),
            pltpu.VMEM((GRP, CHUNK), jnp.int32),
            pltpu.VMEM((GRP, CHUNK), jnp.int32),
            pltpu.SemaphoreType.DMA,
            pltpu.SemaphoreType.DMA,
            pltpu.SemaphoreType.DMA,
            pltpu.SemaphoreType.DMA,
        ),
    )(dstp)


def _tc_layer(h, a0, a1, d0, d1, Ws, Wn, b, act):
    n, din = h.shape
    dout = Ws.shape[1]
    blk = 1000

    def body(h_ref, a0_ref, a1_ref, d0_ref, d1_ref, ws_ref, wn_ref, b_ref,
             o_ref):
        deg = d0_ref[:, 0:1] + d1_ref[:, 0:1]
        inv = 1.0 / jnp.maximum(deg, 1.0)
        hn = (a0_ref[...] + a1_ref[...]) * inv
        out = jnp.dot(h_ref[...], ws_ref[...],
                      preferred_element_type=jnp.float32)
        out = out + jnp.dot(hn, wn_ref[...],
                            preferred_element_type=jnp.float32)
        out = out + b_ref[...]
        if act:
            out = jnp.maximum(out, 0.0)
        o_ref[...] = out

    return pl.pallas_call(
        body,
        grid=(n // blk,),
        in_specs=[
            pl.BlockSpec((blk, din), lambda i: (i, 0)),
            pl.BlockSpec((blk, din), lambda i: (i, 0)),
            pl.BlockSpec((blk, din), lambda i: (i, 0)),
            pl.BlockSpec((blk, F), lambda i: (i, 0)),
            pl.BlockSpec((blk, F), lambda i: (i, 0)),
            pl.BlockSpec((din, dout), lambda i: (0, 0)),
            pl.BlockSpec((din, dout), lambda i: (0, 0)),
            pl.BlockSpec((1, dout), lambda i: (0, 0)),
        ],
        out_specs=pl.BlockSpec((blk, dout), lambda i: (i, 0)),
        out_shape=jax.ShapeDtypeStruct((n, dout), jnp.float32),
    )(h, a0, a1, d0, d1, Ws, Wn, b.reshape(1, dout))


def kernel(features, edge_index, Ws0, Wn0, b0, Ws1, Wn1, b1, Ws2, Wn2, b2):
    src = edge_index[0]
    dst = edge_index[1]
    srcp, dstp = _pad_edges(src, dst)

    degp = _sc_deg(dstp)
    d0, d1 = degp[0], degp[1]
    agg0 = _sc_agg(features, src, dst)
    h1 = _tc_layer(features, agg0[0], agg0[1], d0, d1, Ws0, Wn0, b0, True)
    agg1 = _sc_agg(h1, src, dst)
    h2 = _tc_layer(h1, agg1[0], agg1[1], d0, d1, Ws1, Wn1, b1, True)
    agg2 = _sc_agg(h2, src, dst)
    return _tc_layer(h2, agg2[0], agg2[1], d0, d1, Ws2, Wn2, b2, False)
